# Initial kernel scaffold; baseline (speedup 1.0000x reference)
import jax
import jax.numpy as jnp
from jax.experimental import pallas as pl


def kernel(x, edge_index, edge_attr, params):
    raise NotImplementedError("write your pallas kernel here")


# speedup vs baseline: 2.0022x; 2.0022x over previous
import functools
import math

import jax
import jax.numpy as jnp
from jax import lax
from jax.experimental import pallas as pl
from jax.experimental.pallas import tpu as pltpu
from jax.experimental.pallas import tpu_sc as plsc

D = 128
NC, NS, L = 2, 16, 16
NW = NC * NS
NEG = -1e30


def _pad_rows(a, m):
    p = (-a.shape[0]) % m
    if p:
        pad = [(0, p)] + [(0, 0)] * (a.ndim - 1)
        a = jnp.pad(a, pad)
    return a


def _matmul_body(a_ref, b_ref, o_ref):
    o_ref[...] = jnp.dot(a_ref[...], b_ref[...],
                         preferred_element_type=jnp.float32)


def tc_matmul(a, b, bm=256):
    m, k = a.shape
    _, n = b.shape
    return pl.pallas_call(
        _matmul_body,
        grid=(m // bm,),
        in_specs=[pl.BlockSpec((bm, k), lambda i: (i, 0)),
                  pl.BlockSpec((k, n), lambda i: (0, 0))],
        out_specs=pl.BlockSpec((bm, n), lambda i: (i, 0)),
        out_shape=jax.ShapeDtypeStruct((m, n), jnp.float32),
    )(a, b)


def _mesh():
    return plsc.VectorSubcoreMesh(core_axis_name="c", subcore_axis_name="s")


def _wid():
    return lax.axis_index("s") * NC + lax.axis_index("c")


def _seg_update(tab, ktmp, vtmp, d16, v16, iota, op_max):
    dk, ev = plsc.sort_key_val(d16, v16)
    for s in (1, 2, 4, 8):
        ktmp[...] = dk
        vtmp[...] = ev
        ids = jnp.maximum(iota - s, 0)
        pk = plsc.load_gather(ktmp, [ids])
        pv = plsc.load_gather(vtmp, [ids])
        ok = (iota >= s) & (pk == dk)
        cmb = jnp.maximum(ev, pv) if op_max else ev + pv
        ev = jnp.where(ok, cmb, ev)
    ktmp[...] = dk
    nk = plsc.load_gather(ktmp, [jnp.minimum(iota + 1, L - 1)])
    last = (nk != dk) | (iota == L - 1)
    cur = plsc.load_gather(tab, [dk])
    newv = jnp.maximum(cur, ev) if op_max else cur + ev
    plsc.store_scatter(tab, [dk], newv, mask=last)


@functools.lru_cache(maxsize=None)
def _edge_conv_kernel(n_pad, e_total):
    et = e_total // NW
    b = 200
    nch = et // b
    rt = n_pad // NS

    @functools.partial(
        pl.kernel,
        out_type=jax.ShapeDtypeStruct((NC, n_pad, D), jnp.float32),
        mesh=_mesh(),
        compiler_params=pltpu.CompilerParams(needs_layout_passes=False),
        scratch_types=[
            pltpu.VMEM((b, D), jnp.float32),
            pltpu.VMEM((b,), jnp.int32),
            pltpu.VMEM((b,), jnp.int32),
            pltpu.VMEM_SHARED((n_pad, D), jnp.float32),
            pltpu.SemaphoreType.DMA,
        ],
    )
    def k(xw, eb, src, dst, zrows, out, rowb, sidx, didx, acc, sem):
        cid = lax.axis_index("c")
        sid = lax.axis_index("s")
        wid = sid * NC + cid
        pltpu.sync_copy(zrows.at[pl.ds(sid * rt, rt)],
                        acc.at[pl.ds(sid * rt, rt)])
        plsc.subcore_barrier()
        base0 = wid * et

        def chunk(ci, _):
            base = base0 + ci * b
            pltpu.sync_copy(src.at[pl.ds(base, b)], sidx)
            pltpu.sync_copy(dst.at[pl.ds(base, b)], didx)
            pltpu.sync_copy(eb.at[pl.ds(base, b)], rowb)
            pltpu.async_copy(xw.at[sidx], rowb, sem, add=True).wait()

            def relu_row(r, _):
                for c in range(D // L):
                    sl = pl.ds(c * L, L)
                    rowb[r, sl] = jnp.maximum(rowb[r, sl], 0.0)
                return 0

            lax.fori_loop(0, b, relu_row, 0)
            pltpu.sync_copy(rowb, acc.at[didx], add=True)
            return 0

        lax.fori_loop(0, nch, chunk, 0)
        plsc.subcore_barrier()
        pltpu.sync_copy(acc.at[pl.ds(sid * rt, rt)],
                        out.at[cid, pl.ds(sid * rt, rt)])

    return k


@functools.lru_cache(maxsize=None)
def _gat_max_kernel(n_pad, e_total):
    et = e_total // NW
    bc = 2000
    nch = et // bc

    @functools.partial(
        pl.kernel,
        out_type=jax.ShapeDtypeStruct((NW, n_pad), jnp.float32),
        mesh=_mesh(),
        compiler_params=pltpu.CompilerParams(needs_layout_passes=False),
        scratch_types=[
            pltpu.VMEM((n_pad,), jnp.float32),
            pltpu.VMEM((n_pad,), jnp.float32),
            pltpu.VMEM((n_pad,), jnp.float32),
            pltpu.VMEM((bc,), jnp.int32),
            pltpu.VMEM((bc,), jnp.int32),
            pltpu.VMEM((bc,), jnp.float32),
            pltpu.VMEM((L,), jnp.int32),
            pltpu.VMEM((L,), jnp.float32),
        ],
    )
    def k(es, ed, src, dst, msk, out, est, edt, mtab, sidx, didx, mbuf,
          ktmp, vtmp):
        wid = _wid()
        pltpu.sync_copy(es, est)
        pltpu.sync_copy(ed, edt)
        iota = lax.iota(jnp.int32, L)
        neg = jnp.full((L,), NEG, jnp.float32)

        def init(i, _):
            mtab[pl.ds(i * L, L)] = neg
            return 0

        lax.fori_loop(0, n_pad // L, init, 0)

        def chunk(ci, _):
            base = wid * et + ci * bc
            pltpu.sync_copy(src.at[pl.ds(base, bc)], sidx)
            pltpu.sync_copy(dst.at[pl.ds(base, bc)], didx)
            pltpu.sync_copy(msk.at[pl.ds(base, bc)], mbuf)

            def step(j, _):
                sl = pl.ds(j * L, L)
                s16 = sidx[sl]
                d16 = didx[sl]
                m16 = mbuf[sl]
                vs = plsc.load_gather(est, [s16])
                vd = plsc.load_gather(edt, [d16])
                e = vs + vd
                e = jnp.where(e >= 0.0, e, 0.2 * e)
                e = jnp.where(m16 > 0.0, e, NEG)
                _seg_update(mtab, ktmp, vtmp, d16, e, iota, True)
                return 0

            lax.fori_loop(0, bc // L, step, 0)
            return 0

        lax.fori_loop(0, nch, chunk, 0)
        pltpu.sync_copy(mtab, out.at[wid])

    return k


@functools.lru_cache(maxsize=None)
def _gat_den_kernel(n_pad, e_total):
    et = e_total // NW
    bc = 2000
    nch = et // bc

    @functools.partial(
        pl.kernel,
        out_type=jax.ShapeDtypeStruct((NW, n_pad), jnp.float32),
        mesh=_mesh(),
        compiler_params=pltpu.CompilerParams(needs_layout_passes=False),
        scratch_types=[
            pltpu.VMEM((n_pad,), jnp.float32),
            pltpu.VMEM((n_pad,), jnp.float32),
            pltpu.VMEM((n_pad,), jnp.float32),
            pltpu.VMEM((n_pad,), jnp.float32),
            pltpu.VMEM((bc,), jnp.int32),
            pltpu.VMEM((bc,), jnp.int32),
            pltpu.VMEM((bc,), jnp.float32),
            pltpu.VMEM((L,), jnp.int32),
            pltpu.VMEM((L,), jnp.float32),
        ],
    )
    def k(es, ed, cmx, src, dst, msk, out, est, edt, ctab, dtab, sidx, didx,
          mbuf, ktmp, vtmp):
        wid = _wid()
        pltpu.sync_copy(es, est)
        pltpu.sync_copy(ed, edt)
        pltpu.sync_copy(cmx, ctab)
        iota = lax.iota(jnp.int32, L)
        zero = jnp.zeros((L,), jnp.float32)

        def init(i, _):
            dtab[pl.ds(i * L, L)] = zero
            return 0

        lax.fori_loop(0, n_pad // L, init, 0)

        def chunk(ci, _):
            base = wid * et + ci * bc
            pltpu.sync_copy(src.at[pl.ds(base, bc)], sidx)
            pltpu.sync_copy(dst.at[pl.ds(base, bc)], didx)
            pltpu.sync_copy(msk.at[pl.ds(base, bc)], mbuf)

            def step(j, _):
                sl = pl.ds(j * L, L)
                s16 = sidx[sl]
                d16 = didx[sl]
                m16 = mbuf[sl]
                vs = plsc.load_gather(est, [s16])
                vd = plsc.load_gather(edt, [d16])
                e = vs + vd
                e = jnp.where(e >= 0.0, e, 0.2 * e)
                vc = plsc.load_gather(ctab, [d16])
                t = jnp.exp(e - vc) * m16
                _seg_update(dtab, ktmp, vtmp, d16, t, iota, False)
                return 0

            lax.fori_loop(0, bc // L, step, 0)
            return 0

        lax.fori_loop(0, nch, chunk, 0)
        pltpu.sync_copy(dtab, out.at[wid])

    return k


@functools.lru_cache(maxsize=None)
def _gat_rows_kernel(n_pad, e_total):
    et = e_total // NW
    b = 80
    nch = et // b
    rt = n_pad // NS

    @functools.partial(
        pl.kernel,
        out_type=jax.ShapeDtypeStruct((NC, n_pad, D), jnp.float32),
        mesh=_mesh(),
        compiler_params=pltpu.CompilerParams(needs_layout_passes=False),
        scratch_types=[
            pltpu.VMEM((n_pad,), jnp.float32),
            pltpu.VMEM((n_pad,), jnp.float32),
            pltpu.VMEM((n_pad,), jnp.float32),
            pltpu.VMEM((b, D), jnp.float32),
            pltpu.VMEM((b,), jnp.int32),
            pltpu.VMEM((b,), jnp.int32),
            pltpu.VMEM((b,), jnp.float32),
            pltpu.VMEM((b,), jnp.float32),
            pltpu.VMEM_SHARED((n_pad, D), jnp.float32),
            pltpu.SemaphoreType.DMA,
        ],
    )
    def k(es, ed, q, h, src, dst, msk, zrows, out, est, edt, qtab,
          rowb, sidx, didx, mbuf, abuf, acc, sem):
        cid = lax.axis_index("c")
        sid = lax.axis_index("s")
        wid = sid * NC + cid
        pltpu.sync_copy(es, est)
        pltpu.sync_copy(ed, edt)
        pltpu.sync_copy(q, qtab)
        pltpu.sync_copy(zrows.at[pl.ds(sid * rt, rt)],
                        acc.at[pl.ds(sid * rt, rt)])
        plsc.subcore_barrier()

        def chunk(ci, _):
            base = wid * et + ci * b
            pltpu.sync_copy(src.at[pl.ds(base, b)], sidx)
            pltpu.sync_copy(dst.at[pl.ds(base, b)], didx)
            pltpu.sync_copy(msk.at[pl.ds(base, b)], mbuf)
            pltpu.async_copy(h.at[sidx], rowb, sem).wait()

            def alpha(j, _):
                sl = pl.ds(j * L, L)
                s16 = sidx[sl]
                d16 = didx[sl]
                m16 = mbuf[sl]
                vs = plsc.load_gather(est, [s16])
                vd = plsc.load_gather(edt, [d16])
                e = vs + vd
                e = jnp.where(e >= 0.0, e, 0.2 * e)
                vq = plsc.load_gather(qtab, [d16])
                abuf[sl] = jnp.exp(e - vq) * m16
                return 0

            lax.fori_loop(0, b // L, alpha, 0)

            def scale(j, _):
                a16 = abuf[pl.ds(j * L, L)]
                for r in range(L):
                    a = a16[r]
                    row = j * L + r
                    for c in range(D // L):
                        sl = pl.ds(c * L, L)
                        rowb[row, sl] = rowb[row, sl] * a
                return 0

            lax.fori_loop(0, b // L, scale, 0)
            pltpu.sync_copy(rowb, acc.at[didx], add=True)
            return 0

        lax.fori_loop(0, nch, chunk, 0)
        plsc.subcore_barrier()
        pltpu.sync_copy(acc.at[pl.ds(sid * rt, rt)],
                        out.at[cid, pl.ds(sid * rt, rt)])

    return k


@functools.lru_cache(maxsize=None)
def _pool_kernel(n_pad, k_real, k_pad, e_total):
    et = e_total // NW
    bc = 2000
    nch = et // bc
    rb = k_pad // NW

    @functools.partial(
        pl.kernel,
        out_type=[
            jax.ShapeDtypeStruct((k_pad, D), jnp.float32),
            jax.ShapeDtypeStruct((e_total,), jnp.int32),
            jax.ShapeDtypeStruct((e_total,), jnp.int32),
            jax.ShapeDtypeStruct((e_total,), jnp.float32),
            jax.ShapeDtypeStruct((n_pad,), jnp.int32),
        ],
        mesh=_mesh(),
        compiler_params=pltpu.CompilerParams(needs_layout_passes=False),
        scratch_types=[
            pltpu.VMEM((n_pad,), jnp.int32),
            pltpu.VMEM((k_pad,), jnp.int32),
            pltpu.VMEM((bc,), jnp.int32),
            pltpu.VMEM((bc,), jnp.int32),
            pltpu.VMEM((bc,), jnp.float32),
            pltpu.VMEM((bc,), jnp.int32),
            pltpu.VMEM((bc,), jnp.int32),
            pltpu.VMEM((bc,), jnp.float32),
            pltpu.VMEM((rb, D), jnp.float32),
            pltpu.VMEM((rb,), jnp.int32),
            pltpu.VMEM((rb,), jnp.float32),
            pltpu.SemaphoreType.DMA,
        ],
    )
    def k(x, perm, scl, src, dst, msk, xn, ns, nd, nmsk, nidx_out, nidx,
          pbuf, sidx, didx, mbuf, nsb, ndb, nmb, rowb, ridx, scb, sem):
        wid = _wid()
        iota = lax.iota(jnp.int32, L)
        minus1 = jnp.full((L,), -1, jnp.int32)

        def init(i, _):
            nidx[pl.ds(i * L, L)] = minus1
            return 0

        lax.fori_loop(0, n_pad // L, init, 0)
        pltpu.sync_copy(perm, pbuf)

        def setidx(j, _):
            sl = pl.ds(j * L, L)
            p16 = pbuf[sl]
            pos = j * L + iota
            plsc.store_scatter(nidx, [p16], pos, mask=pos < k_real)
            return 0

        lax.fori_loop(0, k_pad // L, setidx, 0)

        def chunk(ci, _):
            base = wid * et + ci * bc
            pltpu.sync_copy(src.at[pl.ds(base, bc)], sidx)
            pltpu.sync_copy(dst.at[pl.ds(base, bc)], didx)
            pltpu.sync_copy(msk.at[pl.ds(base, bc)], mbuf)

            def step(j, _):
                sl = pl.ds(j * L, L)
                s16 = sidx[sl]
                d16 = didx[sl]
                m16 = mbuf[sl]
                a = plsc.load_gather(nidx, [s16])
                bb = plsc.load_gather(nidx, [d16])
                valid = (a >= 0) & (bb >= 0) & (m16 > 0.0)
                nsb[sl] = jnp.where(valid, a, 0)
                ndb[sl] = jnp.where(valid, bb, 0)
                nmb[sl] = jnp.where(valid, 1.0, 0.0)
                return 0

            lax.fori_loop(0, bc // L, step, 0)
            pltpu.sync_copy(nsb, ns.at[pl.ds(base, bc)])
            pltpu.sync_copy(ndb, nd.at[pl.ds(base, bc)])
            pltpu.sync_copy(nmb, nmsk.at[pl.ds(base, bc)])
            return 0

        lax.fori_loop(0, nch, chunk, 0)

        r0 = wid * rb
        pltpu.sync_copy(perm.at[pl.ds(r0, rb)], ridx)
        pltpu.async_copy(x.at[ridx], rowb, sem).wait()
        pltpu.sync_copy(scl.at[pl.ds(r0, rb)], scb)

        def scale(j, _):
            a16 = scb[pl.ds(j * L, L)]
            for r in range(L):
                a = a16[r]
                row = j * L + r
                for c in range(D // L):
                    sl = pl.ds(c * L, L)
                    rowb[row, sl] = rowb[row, sl] * a
            return 0

        lax.fori_loop(0, rb // L, scale, 0)
        pltpu.sync_copy(rowb, xn.at[pl.ds(r0, rb)])

        @pl.when(wid == 0)
        def _():
            pltpu.sync_copy(nidx, nidx_out)

    return k


@functools.lru_cache(maxsize=None)
def _unpool_kernel(n_pad, k_pad):
    rt = n_pad // NW

    @functools.partial(
        pl.kernel,
        out_type=jax.ShapeDtypeStruct((n_pad, D), jnp.float32),
        mesh=_mesh(),
        compiler_params=pltpu.CompilerParams(needs_layout_passes=False),
        scratch_types=[
            pltpu.VMEM((rt,), jnp.int32),
            pltpu.VMEM((rt,), jnp.int32),
            pltpu.VMEM((rt, D), jnp.float32),
            pltpu.VMEM((rt, D), jnp.float32),
            pltpu.SemaphoreType.DMA,
        ],
    )
    def k(res, xsm, inv, out, ibuf, cbuf, gbuf, rbuf, sem):
        wid = _wid()
        r0 = wid * rt
        pltpu.sync_copy(inv.at[pl.ds(r0, rt)], ibuf)

        def clamp(j, _):
            sl = pl.ds(j * L, L)
            cbuf[sl] = jnp.maximum(ibuf[sl], 0)
            return 0

        lax.fori_loop(0, rt // L, clamp, 0)
        pltpu.async_copy(xsm.at[cbuf], gbuf, sem).wait()
        pltpu.sync_copy(res.at[pl.ds(r0, rt)], rbuf)

        def comb(j, _):
            w16 = jnp.where(ibuf[pl.ds(j * L, L)] >= 0,
                            jnp.ones((L,), jnp.float32),
                            jnp.zeros((L,), jnp.float32))
            for r in range(L):
                w = w16[r]
                row = j * L + r
                for c in range(D // L):
                    sl = pl.ds(c * L, L)
                    rbuf[row, sl] = rbuf[row, sl] + gbuf[row, sl] * w
            return 0

        lax.fori_loop(0, rt // L, comb, 0)
        pltpu.sync_copy(rbuf, out.at[pl.ds(r0, rt)])

    return k


def _lrelu(x):
    return jnp.where(x >= 0, x, 0.2 * x)


def _gat(xp, n_pad, src, dst, msk, p):
    e_total = src.shape[0]
    wfull = jnp.concatenate(
        [p['W'], (p['W'] @ p['as'])[:, None], (p['W'] @ p['ad'])[:, None]],
        axis=1)
    wfull = jnp.pad(wfull, ((0, 0), (0, 256 - D - 2)))
    hz = tc_matmul(xp, wfull)
    h = hz[:, :D]
    es = hz[:, D]
    ed = hz[:, D + 1]
    e_self = _lrelu(es + ed)
    mparts = _gat_max_kernel(n_pad, e_total)(es, ed, src, dst, msk)
    emax = jnp.maximum(jnp.max(mparts, axis=0), e_self)
    dparts = _gat_den_kernel(n_pad, e_total)(es, ed, emax, src, dst, msk)
    t_self = jnp.exp(e_self - emax)
    den = jnp.sum(dparts, axis=0) + t_self + 1e-16
    rec = 1.0 / den
    q = emax + jnp.log(den)
    zrows = jnp.zeros((n_pad, D), jnp.float32)
    rparts = _gat_rows_kernel(n_pad, e_total)(es, ed, q, h, src, dst,
                                              msk, zrows)
    out = rparts[0] + rparts[1] + (t_self * rec)[:, None] * h + p['b']
    return out


def kernel(x, edge_index, edge_attr, params):
    n0 = x.shape[0]
    e_total = edge_index.shape[1]
    src = edge_index[0]
    dst = edge_index[1]
    msk = jnp.ones((e_total,), jnp.float32)

    n_pad = ((n0 + 255) // 256) * 256
    xp = _pad_rows(x, 256)
    ea_pad = jnp.pad(edge_attr, ((0, 0), (0, 5)))

    for i in range(3):
        p = params['down%d' % i]
        w2 = jnp.concatenate([p['Wm'][:D], p['Wn']], axis=1)
        z = tc_matmul(xp, w2)
        wme = jnp.pad(p['Wm'][D:], ((0, 5), (0, 0)))
        ebm = tc_matmul(ea_pad, wme)
        zrows = jnp.zeros((n_pad, D), jnp.float32)
        parts = _edge_conv_kernel(n_pad, e_total)(z[:, :D], ebm, src, dst,
                                                  zrows)
        xp = z[:, D:] + parts[0] + parts[1] + p['b']
    xp = jnp.maximum(xp, 0.0)

    n_cur = n0
    xs = [xp]
    eds = [(src, dst, msk, n_pad)]
    invs = []
    for i in range(3):
        k_real = int(math.ceil(0.5 * n_cur))
        k_pad = ((k_real + 255) // 256) * 256
        w = params['pool%d' % i]
        wn = w / (jnp.linalg.norm(w) + 1e-16)
        wmat = jnp.pad(wn[:, None], ((0, 0), (0, 127)))
        score = tc_matmul(xp, wmat)[:n_cur, 0]
        vals, perm = jax.lax.top_k(score, k_real)
        scale = jnp.tanh(vals)
        perm_p = jnp.pad(perm, (0, k_pad - k_real)).astype(jnp.int32)
        scale_p = jnp.pad(scale, (0, k_pad - k_real))
        xn, ns, nd, nmsk, nidx = _pool_kernel(n_pad, k_real, k_pad, e_total)(
            xp, perm_p, scale_p, src, dst, msk)
        invs.append(nidx)
        src, dst, msk = ns, nd, nmsk
        n_cur, n_pad, xp = k_real, k_pad, xn
        xp = jnp.maximum(_gat(xp, n_pad, src, dst, msk,
                              params['gdown%d' % i]), 0.0)
        if i < 2:
            xs.append(xp)
            eds.append((src, dst, msk, n_pad))

    for i in range(3):
        j = 2 - i
        res = xs[j]
        src, dst, msk, res_pad = eds[j]
        inv = invs[j]
        xp = _unpool_kernel(res_pad, n_pad)(res, xp, inv)
        n_pad = res_pad
        xp = _gat(xp, n_pad, src, dst, msk, params['gup%d' % i])
        if i < 2:
            xp = jnp.maximum(xp, 0.0)

    return xp[:n0]


# speedup vs baseline: 22.7925x; 11.3835x over previous
import functools
import math

import jax
import jax.numpy as jnp
from jax import lax
from jax.experimental import pallas as pl
from jax.experimental.pallas import tpu as pltpu
from jax.experimental.pallas import tpu_sc as plsc

D = 128
NC, NS, L = 2, 16, 16
NW = NC * NS
NEG = -1e30


def _pad_rows(a, m):
    p = (-a.shape[0]) % m
    if p:
        pad = [(0, p)] + [(0, 0)] * (a.ndim - 1)
        a = jnp.pad(a, pad)
    return a


def _matmul_body(a_ref, b_ref, o_ref):
    o_ref[...] = jnp.dot(a_ref[...], b_ref[...],
                         preferred_element_type=jnp.float32)


def tc_matmul(a, b, bm=256):
    m, k = a.shape
    _, n = b.shape
    return pl.pallas_call(
        _matmul_body,
        grid=(m // bm,),
        in_specs=[pl.BlockSpec((bm, k), lambda i: (i, 0)),
                  pl.BlockSpec((k, n), lambda i: (0, 0))],
        out_specs=pl.BlockSpec((bm, n), lambda i: (i, 0)),
        out_shape=jax.ShapeDtypeStruct((m, n), jnp.float32),
    )(a, b)


def _matvec_body(a_ref, w_ref, o_ref):
    a = a_ref[...].astype(jnp.bfloat16).astype(jnp.float32)
    w = w_ref[0:1, :].astype(jnp.bfloat16).astype(jnp.float32)
    s = jnp.sum(a * w, axis=1, keepdims=True)
    o_ref[...] = jnp.broadcast_to(s, o_ref.shape)


def tc_matvec(a, v, bm=256):
    m, k = a.shape
    w2 = jnp.zeros((8, k), jnp.float32).at[0].set(v)
    out = pl.pallas_call(
        _matvec_body,
        grid=(m // bm,),
        in_specs=[pl.BlockSpec((bm, k), lambda i: (i, 0)),
                  pl.BlockSpec((8, k), lambda i: (0, 0))],
        out_specs=pl.BlockSpec((bm, 128), lambda i: (i, 0)),
        out_shape=jax.ShapeDtypeStruct((m, 128), jnp.float32),
    )(a, w2)
    return out[:, 0]


def _mesh():
    return plsc.VectorSubcoreMesh(core_axis_name="c", subcore_axis_name="s")


def _wid():
    return lax.axis_index("s") * NC + lax.axis_index("c")


def _seg_update(tab, ktmp, vtmp, d16, v16, iota, op_max):
    dk, ev = plsc.sort_key_val(d16, v16)
    for s in (1, 2, 4, 8):
        ktmp[...] = dk
        vtmp[...] = ev
        ids = jnp.maximum(iota - s, 0)
        pk = plsc.load_gather(ktmp, [ids])
        pv = plsc.load_gather(vtmp, [ids])
        ok = (iota >= s) & (pk == dk)
        cmb = jnp.maximum(ev, pv) if op_max else ev + pv
        ev = jnp.where(ok, cmb, ev)
    ktmp[...] = dk
    nk = plsc.load_gather(ktmp, [jnp.minimum(iota + 1, L - 1)])
    last = (nk != dk) | (iota == L - 1)
    cur = plsc.load_gather(tab, [dk])
    newv = jnp.maximum(cur, ev) if op_max else cur + ev
    plsc.store_scatter(tab, [dk], newv, mask=last)


@functools.lru_cache(maxsize=None)
def _edge_conv_kernel(n_pad, e_total):
    et = e_total // NW
    b = 200
    nch = et // b
    rt = n_pad // NS

    @functools.partial(
        pl.kernel,
        out_type=jax.ShapeDtypeStruct((NC, n_pad, D), jnp.float32),
        mesh=_mesh(),
        compiler_params=pltpu.CompilerParams(needs_layout_passes=False),
        scratch_types=[
            pltpu.VMEM((b, D), jnp.float32),
            pltpu.VMEM((b,), jnp.int32),
            pltpu.VMEM((b,), jnp.int32),
            pltpu.VMEM_SHARED((n_pad, D), jnp.float32),
            pltpu.SemaphoreType.DMA,
        ],
    )
    def k(xw, eb, src, dst, zrows, out, rowb, sidx, didx, acc, sem):
        cid = lax.axis_index("c")
        sid = lax.axis_index("s")
        wid = sid * NC + cid
        pltpu.sync_copy(zrows.at[pl.ds(sid * rt, rt)],
                        acc.at[pl.ds(sid * rt, rt)])
        plsc.subcore_barrier()
        base0 = wid * et

        def chunk(ci, _):
            base = base0 + ci * b
            pltpu.sync_copy(src.at[pl.ds(base, b)], sidx)
            pltpu.sync_copy(dst.at[pl.ds(base, b)], didx)
            pltpu.sync_copy(eb.at[pl.ds(base, b)], rowb)
            pltpu.async_copy(xw.at[sidx], rowb, sem, add=True).wait()

            def relu_row(r, _):
                for c in range(D // L):
                    sl = pl.ds(c * L, L)
                    rowb[r, sl] = jnp.maximum(rowb[r, sl], 0.0)
                return 0

            lax.fori_loop(0, b, relu_row, 0)
            pltpu.sync_copy(rowb, acc.at[didx], add=True)
            return 0

        lax.fori_loop(0, nch, chunk, 0)
        plsc.subcore_barrier()
        pltpu.sync_copy(acc.at[pl.ds(sid * rt, rt)],
                        out.at[cid, pl.ds(sid * rt, rt)])

    return k


@functools.lru_cache(maxsize=None)
def _gat_max_kernel(n_pad, e_total):
    et = e_total // NW
    bc = 2000
    nch = et // bc

    @functools.partial(
        pl.kernel,
        out_type=jax.ShapeDtypeStruct((NW, n_pad), jnp.float32),
        mesh=_mesh(),
        compiler_params=pltpu.CompilerParams(needs_layout_passes=False),
        scratch_types=[
            pltpu.VMEM((n_pad,), jnp.float32),
            pltpu.VMEM((n_pad,), jnp.float32),
            pltpu.VMEM((n_pad,), jnp.float32),
            pltpu.VMEM((bc,), jnp.int32),
            pltpu.VMEM((bc,), jnp.int32),
            pltpu.VMEM((L,), jnp.int32),
            pltpu.VMEM((L,), jnp.int32),
            pltpu.VMEM((L,), jnp.float32),
        ],
    )
    def k(es, ed, src, dst, cnts, out, est, edt, mtab, sidx, didx, cbuf,
          ktmp, vtmp):
        wid = _wid()
        pltpu.sync_copy(es, est)
        pltpu.sync_copy(ed, edt)
        pltpu.sync_copy(cnts.at[wid], cbuf)
        cnt = cbuf[...][0]
        iota = lax.iota(jnp.int32, L)
        neg = jnp.full((L,), NEG, jnp.float32)

        def init(i, _):
            mtab[pl.ds(i * L, L)] = neg
            return 0

        lax.fori_loop(0, n_pad // L, init, 0)

        def chunk(ci, _):
            base = wid * et + ci * bc
            pltpu.sync_copy(src.at[pl.ds(base, bc)], sidx)
            pltpu.sync_copy(dst.at[pl.ds(base, bc)], didx)

            def step(j, _):
                sl = pl.ds(j * L, L)
                s16 = sidx[sl]
                d16 = didx[sl]
                gidx = ci * bc + j * L + iota
                vs = plsc.load_gather(est, [s16])
                vd = plsc.load_gather(edt, [d16])
                e = vs + vd
                e = jnp.where(e >= 0.0, e, 0.2 * e)
                e = jnp.where(gidx < cnt, e, NEG)
                _seg_update(mtab, ktmp, vtmp, d16, e, iota, True)
                return 0

            lax.fori_loop(0, bc // L, step, 0)
            return 0

        lax.fori_loop(0, (cnt + bc - 1) // bc, chunk, 0)
        pltpu.sync_copy(mtab, out.at[wid])

    return k


@functools.lru_cache(maxsize=None)
def _gat_den_kernel(n_pad, e_total):
    et = e_total // NW
    bc = 2000
    nch = et // bc

    @functools.partial(
        pl.kernel,
        out_type=jax.ShapeDtypeStruct((NW, n_pad), jnp.float32),
        mesh=_mesh(),
        compiler_params=pltpu.CompilerParams(needs_layout_passes=False),
        scratch_types=[
            pltpu.VMEM((n_pad,), jnp.float32),
            pltpu.VMEM((n_pad,), jnp.float32),
            pltpu.VMEM((n_pad,), jnp.float32),
            pltpu.VMEM((n_pad,), jnp.float32),
            pltpu.VMEM((bc,), jnp.int32),
            pltpu.VMEM((bc,), jnp.int32),
            pltpu.VMEM((L,), jnp.int32),
            pltpu.VMEM((L,), jnp.int32),
            pltpu.VMEM((L,), jnp.float32),
        ],
    )
    def k(es, ed, cmx, src, dst, cnts, out, est, edt, ctab, dtab, sidx, didx,
          cbuf, ktmp, vtmp):
        wid = _wid()
        pltpu.sync_copy(es, est)
        pltpu.sync_copy(ed, edt)
        pltpu.sync_copy(cmx, ctab)
        pltpu.sync_copy(cnts.at[wid], cbuf)
        cnt = cbuf[...][0]
        iota = lax.iota(jnp.int32, L)
        zero = jnp.zeros((L,), jnp.float32)

        def init(i, _):
            dtab[pl.ds(i * L, L)] = zero
            return 0

        lax.fori_loop(0, n_pad // L, init, 0)

        def chunk(ci, _):
            base = wid * et + ci * bc
            pltpu.sync_copy(src.at[pl.ds(base, bc)], sidx)
            pltpu.sync_copy(dst.at[pl.ds(base, bc)], didx)

            def step(j, _):
                sl = pl.ds(j * L, L)
                s16 = sidx[sl]
                d16 = didx[sl]
                gidx = ci * bc + j * L + iota
                vs = plsc.load_gather(est, [s16])
                vd = plsc.load_gather(edt, [d16])
                e = vs + vd
                e = jnp.where(e >= 0.0, e, 0.2 * e)
                vc = plsc.load_gather(ctab, [d16])
                t = jnp.exp(e - vc)
                t = jnp.where(gidx < cnt, t, 0.0)
                _seg_update(dtab, ktmp, vtmp, d16, t, iota, False)
                return 0

            lax.fori_loop(0, bc // L, step, 0)
            return 0

        lax.fori_loop(0, (cnt + bc - 1) // bc, chunk, 0)
        pltpu.sync_copy(dtab, out.at[wid])

    return k


@functools.lru_cache(maxsize=None)
def _gat_rows_kernel(n_pad, e_total):
    et = e_total // NW
    b = 80
    nch = et // b
    rt = n_pad // NS

    @functools.partial(
        pl.kernel,
        out_type=jax.ShapeDtypeStruct((NC, n_pad, D), jnp.float32),
        mesh=_mesh(),
        compiler_params=pltpu.CompilerParams(needs_layout_passes=False),
        scratch_types=[
            pltpu.VMEM((n_pad,), jnp.float32),
            pltpu.VMEM((n_pad,), jnp.float32),
            pltpu.VMEM((n_pad,), jnp.float32),
            pltpu.VMEM((b, D), jnp.float32),
            pltpu.VMEM((b,), jnp.int32),
            pltpu.VMEM((b,), jnp.int32),
            pltpu.VMEM((L,), jnp.int32),
            pltpu.VMEM((b,), jnp.float32),
            pltpu.VMEM_SHARED((n_pad, D), jnp.float32),
            pltpu.SemaphoreType.DMA,
        ],
    )
    def k(es, ed, q, h, src, dst, cnts, zrows, out, est, edt, qtab,
          rowb, sidx, didx, cbuf, abuf, acc, sem):
        cid = lax.axis_index("c")
        sid = lax.axis_index("s")
        wid = sid * NC + cid
        pltpu.sync_copy(es, est)
        pltpu.sync_copy(ed, edt)
        pltpu.sync_copy(q, qtab)
        pltpu.sync_copy(cnts.at[wid], cbuf)
        cnt = cbuf[...][0]
        iota = lax.iota(jnp.int32, L)
        pltpu.sync_copy(zrows.at[pl.ds(sid * rt, rt)],
                        acc.at[pl.ds(sid * rt, rt)])
        plsc.subcore_barrier()

        def chunk(ci, _):
            base = wid * et + ci * b
            pltpu.sync_copy(src.at[pl.ds(base, b)], sidx)
            pltpu.sync_copy(dst.at[pl.ds(base, b)], didx)
            pltpu.async_copy(h.at[sidx], rowb, sem).wait()

            def alpha(j, _):
                sl = pl.ds(j * L, L)
                s16 = sidx[sl]
                d16 = didx[sl]
                gidx = ci * b + j * L + iota
                vs = plsc.load_gather(est, [s16])
                vd = plsc.load_gather(edt, [d16])
                e = vs + vd
                e = jnp.where(e >= 0.0, e, 0.2 * e)
                vq = plsc.load_gather(qtab, [d16])
                a = jnp.exp(e - vq)
                abuf[sl] = jnp.where(gidx < cnt, a, 0.0)
                return 0

            lax.fori_loop(0, b // L, alpha, 0)

            def scale(j, _):
                a16 = abuf[pl.ds(j * L, L)]
                for r in range(L):
                    a = a16[r]
                    row = j * L + r
                    for c in range(D // L):
                        sl = pl.ds(c * L, L)
                        rowb[row, sl] = rowb[row, sl] * a
                return 0

            lax.fori_loop(0, b // L, scale, 0)
            pltpu.sync_copy(rowb, acc.at[didx], add=True)
            return 0

        lax.fori_loop(0, (cnt + b - 1) // b, chunk, 0)
        plsc.subcore_barrier()
        pltpu.sync_copy(acc.at[pl.ds(sid * rt, rt)],
                        out.at[cid, pl.ds(sid * rt, rt)])

    return k


@functools.lru_cache(maxsize=None)
def _pool_kernel(n_pad, k_real, k_pad, e_total):
    et = e_total // NW
    bc = 2000
    nch = et // bc
    rb = k_pad // NW

    @functools.partial(
        pl.kernel,
        out_type=[
            jax.ShapeDtypeStruct((k_pad, D), jnp.float32),
            jax.ShapeDtypeStruct((e_total,), jnp.int32),
            jax.ShapeDtypeStruct((e_total,), jnp.int32),
            jax.ShapeDtypeStruct((NW, L), jnp.int32),
            jax.ShapeDtypeStruct((n_pad,), jnp.int32),
        ],
        mesh=_mesh(),
        compiler_params=pltpu.CompilerParams(needs_layout_passes=False),
        scratch_types=[
            pltpu.VMEM((n_pad,), jnp.int32),
            pltpu.VMEM((k_pad,), jnp.int32),
            pltpu.VMEM((bc,), jnp.int32),
            pltpu.VMEM((bc,), jnp.int32),
            pltpu.VMEM((L,), jnp.int32),
            pltpu.VMEM((et + L,), jnp.int32),
            pltpu.VMEM((et + L,), jnp.int32),
            pltpu.VMEM((rb, D), jnp.float32),
            pltpu.VMEM((rb,), jnp.int32),
            pltpu.VMEM((rb,), jnp.float32),
            pltpu.SemaphoreType.DMA,
        ],
    )
    def k(x, perm, scl, src, dst, cnts, xn, ns, nd, cnts_out, nidx_out, nidx,
          pbuf, sidx, didx, cbuf, nsc, ndc, rowb, ridx, scb, sem):
        wid = _wid()
        iota = lax.iota(jnp.int32, L)
        minus1 = jnp.full((L,), -1, jnp.int32)
        zero16 = jnp.zeros((L,), jnp.int32)

        def init(i, _):
            nidx[pl.ds(i * L, L)] = minus1
            return 0

        lax.fori_loop(0, n_pad // L, init, 0)

        def zeroe(i, _):
            nsc[pl.ds(i * L, L)] = zero16
            ndc[pl.ds(i * L, L)] = zero16
            return 0

        lax.fori_loop(0, (et + L) // L, zeroe, 0)
        pltpu.sync_copy(perm, pbuf)
        pltpu.sync_copy(cnts.at[wid], cbuf)
        cnt_in = cbuf[...][0]

        def setidx(j, _):
            sl = pl.ds(j * L, L)
            p16 = pbuf[sl]
            pos = j * L + iota
            plsc.store_scatter(nidx, [p16], pos, mask=pos < k_real)
            return 0

        lax.fori_loop(0, k_pad // L, setidx, 0)

        def chunk(ci, cur):
            base = wid * et + ci * bc
            pltpu.sync_copy(src.at[pl.ds(base, bc)], sidx)
            pltpu.sync_copy(dst.at[pl.ds(base, bc)], didx)

            def step(j, cur):
                sl = pl.ds(j * L, L)
                s16 = sidx[sl]
                d16 = didx[sl]
                gidx = ci * bc + j * L + iota
                a = plsc.load_gather(nidx, [s16])
                bb = plsc.load_gather(nidx, [d16])
                valid = (a >= 0) & (bb >= 0) & (gidx < cnt_in)
                plsc.store_compressed(nsc.at[pl.ds(cur, L)], a, mask=valid)
                plsc.store_compressed(ndc.at[pl.ds(cur, L)], bb, mask=valid)
                nv = plsc.all_reduce_population_count(valid)[0]
                return cur + nv

            return lax.fori_loop(0, bc // L, step, cur)

        cur = lax.fori_loop(0, (cnt_in + bc - 1) // bc, chunk, 0)
        pltpu.sync_copy(nsc.at[pl.ds(0, et)], ns.at[pl.ds(wid * et, et)])
        pltpu.sync_copy(ndc.at[pl.ds(0, et)], nd.at[pl.ds(wid * et, et)])
        cbuf[...] = jnp.full((L,), cur, jnp.int32)
        pltpu.sync_copy(cbuf, cnts_out.at[wid])

        r0 = wid * rb
        pltpu.sync_copy(perm.at[pl.ds(r0, rb)], ridx)
        pltpu.async_copy(x.at[ridx], rowb, sem).wait()
        pltpu.sync_copy(scl.at[pl.ds(r0, rb)], scb)

        def scale(j, _):
            a16 = scb[pl.ds(j * L, L)]
            for r in range(L):
                a = a16[r]
                row = j * L + r
                for c in range(D // L):
                    sl = pl.ds(c * L, L)
                    rowb[row, sl] = rowb[row, sl] * a
            return 0

        lax.fori_loop(0, rb // L, scale, 0)
        pltpu.sync_copy(rowb, xn.at[pl.ds(r0, rb)])

        @pl.when(wid == 0)
        def _():
            pltpu.sync_copy(nidx, nidx_out)

    return k


@functools.lru_cache(maxsize=None)
def _unpool_kernel(n_pad, k_pad):
    rt = n_pad // NW

    @functools.partial(
        pl.kernel,
        out_type=jax.ShapeDtypeStruct((n_pad, D), jnp.float32),
        mesh=_mesh(),
        compiler_params=pltpu.CompilerParams(needs_layout_passes=False),
        scratch_types=[
            pltpu.VMEM((rt,), jnp.int32),
            pltpu.VMEM((rt,), jnp.int32),
            pltpu.VMEM((rt, D), jnp.float32),
            pltpu.VMEM((rt, D), jnp.float32),
            pltpu.SemaphoreType.DMA,
        ],
    )
    def k(res, xsm, inv, out, ibuf, cbuf, gbuf, rbuf, sem):
        wid = _wid()
        r0 = wid * rt
        pltpu.sync_copy(inv.at[pl.ds(r0, rt)], ibuf)

        def clamp(j, _):
            sl = pl.ds(j * L, L)
            cbuf[sl] = jnp.maximum(ibuf[sl], 0)
            return 0

        lax.fori_loop(0, rt // L, clamp, 0)
        pltpu.async_copy(xsm.at[cbuf], gbuf, sem).wait()
        pltpu.sync_copy(res.at[pl.ds(r0, rt)], rbuf)

        def comb(j, _):
            w16 = jnp.where(ibuf[pl.ds(j * L, L)] >= 0,
                            jnp.ones((L,), jnp.float32),
                            jnp.zeros((L,), jnp.float32))
            for r in range(L):
                w = w16[r]
                row = j * L + r
                for c in range(D // L):
                    sl = pl.ds(c * L, L)
                    rbuf[row, sl] = rbuf[row, sl] + gbuf[row, sl] * w
            return 0

        lax.fori_loop(0, rt // L, comb, 0)
        pltpu.sync_copy(rbuf, out.at[pl.ds(r0, rt)])

    return k


def _lrelu(x):
    return jnp.where(x >= 0, x, 0.2 * x)


def _gat(xp, n_pad, src, dst, cnts, p):
    e_total = src.shape[0]
    h = tc_matmul(xp, p['W'])
    es = tc_matvec(h, p['as'])
    ed = tc_matvec(h, p['ad'])
    e_self = _lrelu(es + ed)
    mparts = _gat_max_kernel(n_pad, e_total)(es, ed, src, dst, cnts)
    emax = jnp.maximum(jnp.max(mparts, axis=0), e_self)
    dparts = _gat_den_kernel(n_pad, e_total)(es, ed, emax, src, dst, cnts)
    t_self = jnp.exp(e_self - emax)
    den = jnp.sum(dparts, axis=0) + t_self + 1e-16
    rec = 1.0 / den
    q = emax + jnp.log(den)
    zrows = jnp.zeros((n_pad, D), jnp.float32)
    rparts = _gat_rows_kernel(n_pad, e_total)(es, ed, q, h, src, dst,
                                              cnts, zrows)
    out = rparts[0] + rparts[1] + (t_self * rec)[:, None] * h + p['b']
    return out


def kernel(x, edge_index, edge_attr, params):
    n0 = x.shape[0]
    e_total = edge_index.shape[1]
    src = edge_index[0]
    dst = edge_index[1]
    cnts = jnp.full((NW, L), e_total // NW, jnp.int32)

    n_pad = ((n0 + 255) // 256) * 256
    xp = _pad_rows(x, 256)
    ea_pad = jnp.pad(edge_attr, ((0, 0), (0, 5)))

    for i in range(3):
        p = params['down%d' % i]
        w2 = jnp.concatenate([p['Wm'][:D], p['Wn']], axis=1)
        z = tc_matmul(xp, w2)
        wme = jnp.pad(p['Wm'][D:], ((0, 5), (0, 0)))
        ebm = tc_matmul(ea_pad, wme)
        zrows = jnp.zeros((n_pad, D), jnp.float32)
        parts = _edge_conv_kernel(n_pad, e_total)(z[:, :D], ebm, src, dst,
                                                  zrows)
        xp = z[:, D:] + parts[0] + parts[1] + p['b']
    xp = jnp.maximum(xp, 0.0)

    n_cur = n0
    xs = [xp]
    eds = [(src, dst, cnts, n_pad)]
    invs = []
    for i in range(3):
        k_real = int(math.ceil(0.5 * n_cur))
        k_pad = ((k_real + 255) // 256) * 256
        w = params['pool%d' % i]
        score = tc_matvec(xp, w)[:n_cur] / (jnp.linalg.norm(w) + 1e-16)
        vals, perm = jax.lax.top_k(score, k_real)
        scale = jnp.tanh(vals)
        perm_p = jnp.pad(perm, (0, k_pad - k_real)).astype(jnp.int32)
        scale_p = jnp.pad(scale, (0, k_pad - k_real))
        xn, ns, nd, ncnts, nidx = _pool_kernel(n_pad, k_real, k_pad, e_total)(
            xp, perm_p, scale_p, src, dst, cnts)
        invs.append(nidx)
        src, dst, cnts = ns, nd, ncnts
        n_cur, n_pad, xp = k_real, k_pad, xn
        xp = jnp.maximum(_gat(xp, n_pad, src, dst, cnts,
                              params['gdown%d' % i]), 0.0)
        if i < 2:
            xs.append(xp)
            eds.append((src, dst, cnts, n_pad))

    for i in range(3):
        j = 2 - i
        res = xs[j]
        src, dst, cnts, res_pad = eds[j]
        inv = invs[j]
        xp = _unpool_kernel(res_pad, n_pad)(res, xp, inv)
        n_pad = res_pad
        xp = _gat(xp, n_pad, src, dst, cnts, params['gup%d' % i])
        if i < 2:
            xp = jnp.maximum(xp, 0.0)

    return xp[:n0]
